# baseline (speedup 1.0000x reference)
import jax
import jax.numpy as jnp
from jax import lax
from jax.experimental import pallas as pl
from jax.experimental.pallas import tpu as pltpu
from jax.experimental.pallas import tpu_sc as plsc

_B = 4
_S = 2048
_D = 1024

_info = plsc.get_sparse_core_info()
_NC = _info.num_cores
_NS = _info.num_subcores
_NW = _NC * _NS
_SPW = _S // _NW
_C = 8
_NSUP = _SPW // _C
_NG = 3


def _make_sc_kernel():
    mesh = plsc.VectorSubcoreMesh(core_axis_name="c", subcore_axis_name="s")

    def kfn(table_hbm, x_hbm, pos_hbm, out_hbm, idx_v, bufs, gsems, ssems):
        pbuf = lambda gg: bufs.at[gg * (1 + _B)]
        abuf = lambda gg, b: bufs.at[gg * (1 + _B) + 1 + b]

        wid = lax.axis_index("s") * _NC + lax.axis_index("c")
        s0 = wid * _SPW
        for b in range(_B):
            pltpu.sync_copy(
                x_hbm.at[b, pl.ds(s0, _SPW)],
                idx_v.at[pl.ds(b * _SPW, _SPW)],
            )

        def gather_descs(g):
            gg = g % _NG
            descs = [
                pltpu.make_async_copy(
                    pos_hbm.at[pl.ds(s0 + g * _C, _C)], pbuf(gg), gsems.at[gg]
                )
            ]
            for b in range(_B):
                descs.append(
                    pltpu.make_async_copy(
                        table_hbm.at[idx_v.at[pl.ds(b * _SPW + g * _C, _C)]],
                        abuf(gg, b),
                        gsems.at[gg],
                    )
                )
            return descs

        def store_descs(g):
            gg = g % _NG
            return [
                pltpu.make_async_copy(
                    abuf(gg, b),
                    out_hbm.at[b, pl.ds(s0 + g * _C, _C)],
                    ssems.at[gg],
                )
                for b in range(_B)
            ]

        def issue(g):
            if g >= _NG:
                for d in store_descs(g - _NG):
                    d.wait()
            for d in gather_descs(g):
                d.start()

        def consume(g):
            gg = g % _NG
            for d in gather_descs(g):
                d.wait()
            p, bs = pbuf(gg), [abuf(gg, b) for b in range(_B)]

            @plsc.parallel_loop(0, (_C * _D) // 16, unroll=4)
            def add_body(i, _p=p, _bs=bs):
                r = lax.shift_right_logical(i, 6)
                sl = pl.ds((i & 63) * 16, 16)
                pv = _p[r, sl]
                for buf in _bs:
                    buf[r, sl] = buf[r, sl] + pv

            for d in store_descs(g):
                d.start()

        issue(0)
        issue(1)
        for g in range(_NSUP):
            if g + 2 < _NSUP:
                issue(g + 2)
            consume(g)
        for g in range(_NSUP - _NG, _NSUP):
            for d in store_descs(g):
                d.wait()

    return pl.kernel(
        kfn,
        mesh=mesh,
        out_type=jax.ShapeDtypeStruct((_B, _S, _D), jnp.float32),
        scratch_types=[
            pltpu.VMEM((_B * _SPW,), jnp.int32),
            pltpu.VMEM((_NG * (1 + _B), _C, _D), jnp.float32),
            pltpu.SemaphoreType.DMA((_NG,)),
            pltpu.SemaphoreType.DMA((_NG,)),
        ],
    )


_sc_kernel = _make_sc_kernel()


@jax.jit
def kernel(x, table, pos_enc):
    return _sc_kernel(table, x, pos_enc)
